# baseline (speedup 1.0000x reference)
import jax
import jax.numpy as jnp
from jax import lax
from jax.experimental import pallas as pl
from jax.experimental.pallas import tpu as pltpu
from jax.experimental.pallas import tpu_sc as plsc

BT = 128
NT = 40
R = NT * BT
NW = 32


def _gate_body(x_ref, Wgn_ref, bgn_ref, nz_ref,
               pdst_ref, s0m_ref, s1m_ref, ga_ref):
    T, D = x_ref.shape
    E = 8
    xt = x_ref[...]
    hgn = jnp.dot(xt, Wgn_ref[...],
                  preferred_element_type=jnp.float32) + bgn_ref[...]
    hg = hgn[:, :E]
    hn = hgn[:, E:]
    sp = jnp.maximum(hn, 0.0) + jnp.log1p(jnp.exp(-jnp.abs(hn)))
    Hx = hg + nz_ref[...] * sp

    ii = lax.broadcasted_iota(jnp.int32, (T, E), 1)
    m1 = jnp.max(Hx, axis=1, keepdims=True)
    i1 = jnp.min(jnp.where(Hx == m1, ii, E), axis=1, keepdims=True)
    msk1 = ii == i1
    Hx2 = jnp.where(msk1, -jnp.inf, Hx)
    m2 = jnp.max(Hx2, axis=1, keepdims=True)
    i2 = jnp.min(jnp.where(Hx2 == m2, ii, E), axis=1, keepdims=True)
    msk2 = ii == i2
    s1 = 1.0 / (1.0 + jnp.exp(m2 - m1))
    s2 = 1.0 - s1

    sel = (msk1 | msk2).astype(jnp.float32)
    SB = 256
    ri = lax.broadcasted_iota(jnp.int32, (SB, SB), 0)
    ci = lax.broadcasted_iota(jnp.int32, (SB, SB), 1)
    Lt = (ri > ci).astype(jnp.float32)
    nb = T // SB
    bsums = [jnp.sum(sel[b * SB:(b + 1) * SB], axis=0, keepdims=True)
             for b in range(nb)]
    bases = [jnp.zeros((1, E), jnp.float32)]
    for b in range(nb - 1):
        bases.append(bases[b] + bsums[b])
    blocks = [bases[b] + jnp.dot(Lt, sel[b * SB:(b + 1) * SB],
                                 preferred_element_type=jnp.float32)
              for b in range(nb)]
    cnt_before = jnp.concatenate(blocks, axis=0)
    cnt = bases[nb - 1] + bsums[nb - 1]

    pcnt = jnp.ceil(cnt * (1.0 / BT)) * BT
    pend = pcnt
    for sh in (1, 2, 4):
        pend = pend + jnp.concatenate(
            [jnp.zeros((1, sh), jnp.float32), pend[:, :E - sh]], axis=1)
    poff = pend - pcnt

    pos0 = jnp.sum(jnp.where(msk1, cnt_before + poff, 0.0),
                   axis=1, keepdims=True)
    pos1 = jnp.sum(jnp.where(msk2, cnt_before + poff, 0.0),
                   axis=1, keepdims=True)

    pdst_ref[...] = jnp.concatenate([pos0, pos1], axis=1).astype(jnp.int32)
    s0m_ref[...] = jnp.broadcast_to(s1, (T, 16))
    s1m_ref[...] = jnp.broadcast_to(s2, (T, 16))

    tend = pend * (1.0 / BT)
    eye = (lax.broadcasted_iota(jnp.int32, (E, E), 0) ==
           lax.broadcasted_iota(jnp.int32, (E, E), 1)).astype(jnp.float32)
    tendT = lax.dot_general(eye, tend, (((1,), (1,)), ((), ())),
                            preferred_element_type=jnp.float32)
    ts = lax.broadcasted_iota(jnp.int32, (E, 64), 1).astype(jnp.float32)
    gid = jnp.minimum(jnp.sum((ts >= tendT).astype(jnp.float32),
                              axis=0, keepdims=True), float(E - 1))
    actv = (ts[:1, :] < tendT[E - 1, 0]).astype(jnp.float32)
    rowi = lax.broadcasted_iota(jnp.int32, (E, 64), 0)
    gidact = jnp.where(rowi == 0, jnp.broadcast_to(gid, (E, 64)),
                       jnp.where(rowi == 1, jnp.broadcast_to(actv, (E, 64)),
                                 0.0))
    ga_ref[...] = gidact.astype(jnp.int32)


def _dispatch_body(x_h, pdst0_h, pdst1_h, xs_h, idx0_v, idx1_v, rows_v, sem):
    nc = plsc.get_sparse_core_info().num_cores
    wid = lax.axis_index("s") * nc + lax.axis_index("c")
    for c in range(2):
        off = wid * 64 + c * 32
        pltpu.sync_copy(pdst0_h.at[pl.ds(off, 32)], idx0_v)
        pltpu.sync_copy(pdst1_h.at[pl.ds(off, 32)], idx1_v)
        pltpu.sync_copy(x_h.at[pl.ds(off, 32)], rows_v)
        h0 = pltpu.async_copy(rows_v, xs_h.at[idx0_v], sem)
        h1 = pltpu.async_copy(rows_v, xs_h.at[idx1_v], sem)
        h0.wait()
        h1.wait()


def _ffn_body(g_ref, xs_ref, W1_ref, b1_ref, W2_ref, b2_ref, ys_ref,
              W1b_s, W2b_s, s10, s20, s11, s21):
    r = pl.program_id(0)
    a = g_ref[1, r]
    rs = g_ref[2, r]
    par = g_ref[3, r]
    fe = g_ref[4, r]
    fv = g_ref[5, r]

    def fetch(e_id, slot, sem1, sem2):
        pltpu.make_async_copy(W1_ref.at[e_id], W1b_s.at[slot], sem1).start()
        pltpu.make_async_copy(W2_ref.at[e_id], W2b_s.at[slot], sem2).start()

    def drain(slot, sem1, sem2):
        pltpu.make_async_copy(W1_ref.at[0], W1b_s.at[slot], sem1).wait()
        pltpu.make_async_copy(W2_ref.at[0], W2b_s.at[slot], sem2).wait()

    @pl.when(r == 0)
    def _():
        fetch(g_ref[6, 0], 0, s10, s20)

        @pl.when(g_ref[6, 2] > 1)
        def _():
            fetch(g_ref[6, 1], 1, s11, s21)

    @pl.when((rs == 1) & (par == 0))
    def _():
        drain(0, s10, s20)

    @pl.when((rs == 1) & (par == 1))
    def _():
        drain(1, s11, s21)

    @pl.when((rs == 1) & (fv == 1) & (par == 0))
    def _():
        fetch(fe, 1, s11, s21)

    @pl.when((rs == 1) & (fv == 1) & (par == 1))
    def _():
        fetch(fe, 0, s10, s20)

    @pl.when((a == 1) & (par == 0))
    def _():
        h = jnp.dot(xs_ref[...], W1b_s[0],
                    preferred_element_type=jnp.float32) + b1_ref[0]
        h = jnp.maximum(h, 0.0)
        ys_ref[...] = jnp.dot(h, W2b_s[0],
                              preferred_element_type=jnp.float32) + b2_ref[0]

    @pl.when((a == 1) & (par == 1))
    def _():
        h = jnp.dot(xs_ref[...], W1b_s[1],
                    preferred_element_type=jnp.float32) + b1_ref[0]
        h = jnp.maximum(h, 0.0)
        ys_ref[...] = jnp.dot(h, W2b_s[1],
                              preferred_element_type=jnp.float32) + b2_ref[0]


def _combine_body(ys_h, pdst0_h, pdst1_h, s0m_h, s1m_h, res_h,
                  idx0_v, idx1_v, s0m_v, s1m_v, buf0_v, buf1_v, obuf_v, sem):
    nc = plsc.get_sparse_core_info().num_cores
    wid = lax.axis_index("s") * nc + lax.axis_index("c")
    D = 768
    for c in range(2):
        off = wid * 64 + c * 32
        pltpu.sync_copy(pdst0_h.at[pl.ds(off, 32)], idx0_v)
        pltpu.sync_copy(pdst1_h.at[pl.ds(off, 32)], idx1_v)
        pltpu.sync_copy(s0m_h.at[pl.ds(off, 32)], s0m_v)
        pltpu.sync_copy(s1m_h.at[pl.ds(off, 32)], s1m_v)
        h0 = pltpu.async_copy(ys_h.at[idx0_v], buf0_v, sem)
        h1 = pltpu.async_copy(ys_h.at[idx1_v], buf1_v, sem)
        h0.wait()
        h1.wait()

        def cj(j, cr):
            w0 = s0m_v[j, :]
            w1 = s1m_v[j, :]
            for v in range(D // 16):
                sl = pl.ds(v * 16, 16)
                obuf_v[j, sl] = w0 * buf0_v[j, sl] + w1 * buf1_v[j, sl]
            return cr
        lax.fori_loop(0, 32, cj, 0)
        pltpu.sync_copy(obuf_v, res_h.at[pl.ds(off, 32)])


def kernel(x, Wg, bg, Wn, bn, W1, b1, W2, b2):
    B, T, D = x.shape
    E = Wg.shape[1]
    FF = W1.shape[2]

    noise = jax.random.normal(jax.random.PRNGKey(42), shape=(B, T, E),
                              dtype=jnp.float32)
    x2 = x.reshape(T, D)
    nz2 = noise.reshape(T, E)
    Wgn = jnp.concatenate([Wg, Wn], axis=1)
    bgn = jnp.concatenate([bg, bn]).reshape(1, 2 * E)
    b1r = b1.reshape(E, 1, FF)
    b2r = b2.reshape(E, 1, D)

    pdstm, s0m, s1m, gidact = pl.pallas_call(
        _gate_body,
        in_specs=[
            pl.BlockSpec((T, D), lambda: (0, 0)),
            pl.BlockSpec((D, 2 * E), lambda: (0, 0)),
            pl.BlockSpec((1, 2 * E), lambda: (0, 0)),
            pl.BlockSpec((T, E), lambda: (0, 0)),
        ],
        out_specs=[
            pl.BlockSpec((T, 2), lambda: (0, 0)),
            pl.BlockSpec((T, 16), lambda: (0, 0)),
            pl.BlockSpec((T, 16), lambda: (0, 0)),
            pl.BlockSpec((8, 64), lambda: (0, 0)),
        ],
        out_shape=[
            jax.ShapeDtypeStruct((T, 2), jnp.int32),
            jax.ShapeDtypeStruct((T, 16), jnp.float32),
            jax.ShapeDtypeStruct((T, 16), jnp.float32),
            jax.ShapeDtypeStruct((8, 64), jnp.int32),
        ],
    )(x2, Wgn, bgn, nz2)

    pdst0 = pdstm[:, 0]
    pdst1 = pdstm[:, 1]

    mesh = plsc.VectorSubcoreMesh(core_axis_name="c", subcore_axis_name="s")

    xs = pl.kernel(
        _dispatch_body,
        out_type=jax.ShapeDtypeStruct((R, D), jnp.float32),
        mesh=mesh,
        scratch_types=[
            pltpu.VMEM((32,), jnp.int32),
            pltpu.VMEM((32,), jnp.int32),
            pltpu.VMEM((32, D), jnp.float32),
            pltpu.SemaphoreType.DMA,
        ],
    )(x2, pdst0, pdst1)

    gid = gidact[0, :NT]
    actv = gidact[1, :NT]
    rs = jnp.concatenate([jnp.ones((1,), jnp.int32),
                          (gid[1:] != gid[:-1]).astype(jnp.int32) *
                          actv[1:]])
    run_id = jnp.cumsum(rs) - 1
    par = run_id % 2
    rs_pos = jnp.nonzero(rs, size=NT, fill_value=NT - 1)[0]
    re = gid[rs_pos]
    nr = jnp.sum(rs)
    nxt = run_id + 1
    fe = re[jnp.minimum(nxt, NT - 1)]
    fv = ((nxt < nr) & (rs == 1)).astype(jnp.int32)
    fv = fv.at[0].set(0)
    pad = jnp.zeros((64 - NT,), jnp.int32)
    row6 = jnp.zeros((64,), jnp.int32).at[0].set(re[0]).at[1].set(
        re[1]).at[2].set(nr)
    P = jnp.stack([
        jnp.concatenate([gid, pad]),
        jnp.concatenate([actv, pad]),
        jnp.concatenate([rs, pad]),
        jnp.concatenate([par.astype(jnp.int32), pad]),
        jnp.concatenate([fe, pad]),
        jnp.concatenate([fv, pad]),
        row6,
        jnp.zeros((64,), jnp.int32),
    ])

    ys = pl.pallas_call(
        _ffn_body,
        grid_spec=pltpu.PrefetchScalarGridSpec(
            num_scalar_prefetch=1,
            grid=(NT,),
            in_specs=[
                pl.BlockSpec((BT, D), lambda r, g: (r, 0)),
                pl.BlockSpec(memory_space=pltpu.HBM),
                pl.BlockSpec((1, 1, FF), lambda r, g: (g[0, r], 0, 0)),
                pl.BlockSpec(memory_space=pltpu.HBM),
                pl.BlockSpec((1, 1, D), lambda r, g: (g[0, r], 0, 0)),
            ],
            out_specs=pl.BlockSpec((BT, D), lambda r, g: (r, 0)),
            scratch_shapes=[
                pltpu.VMEM((2, D, FF), jnp.float32),
                pltpu.VMEM((2, FF, D), jnp.float32),
                pltpu.SemaphoreType.DMA,
                pltpu.SemaphoreType.DMA,
                pltpu.SemaphoreType.DMA,
                pltpu.SemaphoreType.DMA,
            ],
        ),
        out_shape=jax.ShapeDtypeStruct((R, D), jnp.float32),
        compiler_params=pltpu.CompilerParams(
            dimension_semantics=("arbitrary",),
        ),
    )(P, xs, W1, b1r, W2, b2r)

    res = pl.kernel(
        _combine_body,
        out_type=jax.ShapeDtypeStruct((T, D), jnp.float32),
        mesh=mesh,
        scratch_types=[
            pltpu.VMEM((32,), jnp.int32),
            pltpu.VMEM((32,), jnp.int32),
            pltpu.VMEM((32, 16), jnp.float32),
            pltpu.VMEM((32, 16), jnp.float32),
            pltpu.VMEM((32, D), jnp.float32),
            pltpu.VMEM((32, D), jnp.float32),
            pltpu.VMEM((32, D), jnp.float32),
            pltpu.SemaphoreType.DMA,
        ],
    )(ys, pdst0, pdst1, s0m, s1m)

    return res.reshape(B, T, D)


# speedup vs baseline: 1.1282x; 1.1282x over previous
import jax
import jax.numpy as jnp
from jax import lax
from jax.experimental import pallas as pl
from jax.experimental.pallas import tpu as pltpu
from jax.experimental.pallas import tpu_sc as plsc

BT = 256
NT = 25
R = NT * BT
NW = 32


def _gate_body(x_ref, Wgn_ref, bgn_ref, nz_ref,
               pdst_ref, s0m_ref, s1m_ref, ga_ref):
    T, D = x_ref.shape
    E = 8
    xt = x_ref[...]
    hgn = jnp.dot(xt, Wgn_ref[...],
                  preferred_element_type=jnp.float32) + bgn_ref[...]
    hg = hgn[:, :E]
    hn = hgn[:, E:]
    sp = jnp.maximum(hn, 0.0) + jnp.log1p(jnp.exp(-jnp.abs(hn)))
    Hx = hg + nz_ref[...] * sp

    ii = lax.broadcasted_iota(jnp.int32, (T, E), 1)
    m1 = jnp.max(Hx, axis=1, keepdims=True)
    i1 = jnp.min(jnp.where(Hx == m1, ii, E), axis=1, keepdims=True)
    msk1 = ii == i1
    Hx2 = jnp.where(msk1, -jnp.inf, Hx)
    m2 = jnp.max(Hx2, axis=1, keepdims=True)
    i2 = jnp.min(jnp.where(Hx2 == m2, ii, E), axis=1, keepdims=True)
    msk2 = ii == i2
    s1 = 1.0 / (1.0 + jnp.exp(m2 - m1))
    s2 = 1.0 - s1

    sel = (msk1 | msk2).astype(jnp.float32)
    SB = 256
    ri = lax.broadcasted_iota(jnp.int32, (SB, SB), 0)
    ci = lax.broadcasted_iota(jnp.int32, (SB, SB), 1)
    Lt = (ri > ci).astype(jnp.float32)
    nb = T // SB
    bsums = [jnp.sum(sel[b * SB:(b + 1) * SB], axis=0, keepdims=True)
             for b in range(nb)]
    bases = [jnp.zeros((1, E), jnp.float32)]
    for b in range(nb - 1):
        bases.append(bases[b] + bsums[b])
    blocks = [bases[b] + jnp.dot(Lt, sel[b * SB:(b + 1) * SB],
                                 preferred_element_type=jnp.float32)
              for b in range(nb)]
    cnt_before = jnp.concatenate(blocks, axis=0)
    cnt = bases[nb - 1] + bsums[nb - 1]

    pcnt = jnp.ceil(cnt * (1.0 / BT)) * BT
    pend = pcnt
    for sh in (1, 2, 4):
        pend = pend + jnp.concatenate(
            [jnp.zeros((1, sh), jnp.float32), pend[:, :E - sh]], axis=1)
    poff = pend - pcnt

    pos0 = jnp.sum(jnp.where(msk1, cnt_before + poff, 0.0),
                   axis=1, keepdims=True)
    pos1 = jnp.sum(jnp.where(msk2, cnt_before + poff, 0.0),
                   axis=1, keepdims=True)

    pdst_ref[...] = jnp.concatenate([pos0, pos1], axis=1).astype(jnp.int32)
    s0m_ref[...] = jnp.broadcast_to(s1, (T, 16))
    s1m_ref[...] = jnp.broadcast_to(s2, (T, 16))

    tend = pend * (1.0 / 256.0)
    eye = (lax.broadcasted_iota(jnp.int32, (E, E), 0) ==
           lax.broadcasted_iota(jnp.int32, (E, E), 1)).astype(jnp.float32)
    tendT = lax.dot_general(eye, tend, (((1,), (1,)), ((), ())),
                            preferred_element_type=jnp.float32)
    ts = lax.broadcasted_iota(jnp.int32, (E, 32), 1).astype(jnp.float32)
    gid = jnp.minimum(jnp.sum((ts >= tendT).astype(jnp.float32),
                              axis=0, keepdims=True), float(E - 1))
    actv = (ts[:1, :] < tendT[E - 1, 0]).astype(jnp.float32)
    rowi = lax.broadcasted_iota(jnp.int32, (E, 32), 0)
    gidact = jnp.where(rowi == 0, jnp.broadcast_to(gid, (E, 32)),
                       jnp.where(rowi == 1, jnp.broadcast_to(actv, (E, 32)),
                                 0.0))
    ga_ref[...] = gidact.astype(jnp.int32)


def _dispatch_body(x_h, pdst0_h, pdst1_h, xs_h, idx0_v, idx1_v, rows_v, sem):
    nc = plsc.get_sparse_core_info().num_cores
    wid = lax.axis_index("s") * nc + lax.axis_index("c")
    for c in range(2):
        off = wid * 64 + c * 32
        pltpu.sync_copy(pdst0_h.at[pl.ds(off, 32)], idx0_v)
        pltpu.sync_copy(pdst1_h.at[pl.ds(off, 32)], idx1_v)
        pltpu.sync_copy(x_h.at[pl.ds(off, 32)], rows_v)
        h0 = pltpu.async_copy(rows_v, xs_h.at[idx0_v], sem)
        h1 = pltpu.async_copy(rows_v, xs_h.at[idx1_v], sem)
        h0.wait()
        h1.wait()


def _ffn_body(g_ref, xs_ref, W1_ref, b1_ref, W2_ref, b2_ref, ys_ref,
              W1b_s, W2b_s, s10, s20, s11, s21):
    r = pl.program_id(0)
    a = g_ref[1, r]
    rs = g_ref[2, r]
    par = g_ref[3, r]
    fe = g_ref[4, r]
    fv = g_ref[5, r]

    def fetch(e_id, slot, sem1, sem2):
        pltpu.make_async_copy(W1_ref.at[e_id], W1b_s.at[slot], sem1).start()
        pltpu.make_async_copy(W2_ref.at[e_id], W2b_s.at[slot], sem2).start()

    def drain(slot, sem1, sem2):
        pltpu.make_async_copy(W1_ref.at[0], W1b_s.at[slot], sem1).wait()
        pltpu.make_async_copy(W2_ref.at[0], W2b_s.at[slot], sem2).wait()

    @pl.when(r == 0)
    def _():
        fetch(g_ref[6, 0], 0, s10, s20)

        @pl.when(g_ref[6, 2] > 1)
        def _():
            fetch(g_ref[6, 1], 1, s11, s21)

    @pl.when((rs == 1) & (par == 0))
    def _():
        drain(0, s10, s20)

    @pl.when((rs == 1) & (par == 1))
    def _():
        drain(1, s11, s21)

    @pl.when((rs == 1) & (fv == 1) & (par == 0))
    def _():
        fetch(fe, 1, s11, s21)

    @pl.when((rs == 1) & (fv == 1) & (par == 1))
    def _():
        fetch(fe, 0, s10, s20)

    @pl.when((a == 1) & (par == 0))
    def _():
        h = jnp.dot(xs_ref[...], W1b_s[0],
                    preferred_element_type=jnp.float32) + b1_ref[0]
        h = jnp.maximum(h, 0.0)
        ys_ref[...] = jnp.dot(h, W2b_s[0],
                              preferred_element_type=jnp.float32) + b2_ref[0]

    @pl.when((a == 1) & (par == 1))
    def _():
        h = jnp.dot(xs_ref[...], W1b_s[1],
                    preferred_element_type=jnp.float32) + b1_ref[0]
        h = jnp.maximum(h, 0.0)
        ys_ref[...] = jnp.dot(h, W2b_s[1],
                              preferred_element_type=jnp.float32) + b2_ref[0]


def _combine_body(ys_h, pdst0_h, pdst1_h, s0m_h, s1m_h, res_h,
                  idx0_v, idx1_v, s0m_v, s1m_v, buf0_v, buf1_v, obuf_v, sem):
    nc = plsc.get_sparse_core_info().num_cores
    wid = lax.axis_index("s") * nc + lax.axis_index("c")
    D = 768
    for c in range(2):
        off = wid * 64 + c * 32
        pltpu.sync_copy(pdst0_h.at[pl.ds(off, 32)], idx0_v)
        pltpu.sync_copy(pdst1_h.at[pl.ds(off, 32)], idx1_v)
        pltpu.sync_copy(s0m_h.at[pl.ds(off, 32)], s0m_v)
        pltpu.sync_copy(s1m_h.at[pl.ds(off, 32)], s1m_v)
        h0 = pltpu.async_copy(ys_h.at[idx0_v], buf0_v, sem)
        h1 = pltpu.async_copy(ys_h.at[idx1_v], buf1_v, sem)
        h0.wait()
        h1.wait()

        def cj(j, cr):
            w0 = s0m_v[j, :]
            w1 = s1m_v[j, :]
            for v in range(D // 16):
                sl = pl.ds(v * 16, 16)
                obuf_v[j, sl] = w0 * buf0_v[j, sl] + w1 * buf1_v[j, sl]
            return cr
        lax.fori_loop(0, 32, cj, 0)
        pltpu.sync_copy(obuf_v, res_h.at[pl.ds(off, 32)])


def kernel(x, Wg, bg, Wn, bn, W1, b1, W2, b2):
    B, T, D = x.shape
    E = Wg.shape[1]
    FF = W1.shape[2]

    noise = jax.random.normal(jax.random.PRNGKey(42), shape=(B, T, E),
                              dtype=jnp.float32)
    x2 = x.reshape(T, D)
    nz2 = noise.reshape(T, E)
    Wgn = jnp.concatenate([Wg, Wn], axis=1)
    bgn = jnp.concatenate([bg, bn]).reshape(1, 2 * E)
    b1r = b1.reshape(E, 1, FF)
    b2r = b2.reshape(E, 1, D)

    pdstm, s0m, s1m, gidact = pl.pallas_call(
        _gate_body,
        in_specs=[
            pl.BlockSpec((T, D), lambda: (0, 0)),
            pl.BlockSpec((D, 2 * E), lambda: (0, 0)),
            pl.BlockSpec((1, 2 * E), lambda: (0, 0)),
            pl.BlockSpec((T, E), lambda: (0, 0)),
        ],
        out_specs=[
            pl.BlockSpec((T, 2), lambda: (0, 0)),
            pl.BlockSpec((T, 16), lambda: (0, 0)),
            pl.BlockSpec((T, 16), lambda: (0, 0)),
            pl.BlockSpec((8, 32), lambda: (0, 0)),
        ],
        out_shape=[
            jax.ShapeDtypeStruct((T, 2), jnp.int32),
            jax.ShapeDtypeStruct((T, 16), jnp.float32),
            jax.ShapeDtypeStruct((T, 16), jnp.float32),
            jax.ShapeDtypeStruct((8, 32), jnp.int32),
        ],
    )(x2, Wgn, bgn, nz2)

    pdst0 = pdstm[:, 0]
    pdst1 = pdstm[:, 1]

    mesh = plsc.VectorSubcoreMesh(core_axis_name="c", subcore_axis_name="s")

    xs = pl.kernel(
        _dispatch_body,
        out_type=jax.ShapeDtypeStruct((R, D), jnp.float32),
        mesh=mesh,
        scratch_types=[
            pltpu.VMEM((32,), jnp.int32),
            pltpu.VMEM((32,), jnp.int32),
            pltpu.VMEM((32, D), jnp.float32),
            pltpu.SemaphoreType.DMA,
        ],
    )(x2, pdst0, pdst1)

    gid = gidact[0, :NT]
    actv = gidact[1, :NT]
    rs = jnp.concatenate([jnp.ones((1,), jnp.int32),
                          (gid[1:] != gid[:-1]).astype(jnp.int32) *
                          actv[1:]])
    run_id = jnp.cumsum(rs) - 1
    par = run_id % 2
    rs_pos = jnp.nonzero(rs, size=NT, fill_value=NT - 1)[0]
    re = gid[rs_pos]
    nr = jnp.sum(rs)
    nxt = run_id + 1
    fe = re[jnp.minimum(nxt, NT - 1)]
    fv = ((nxt < nr) & (rs == 1)).astype(jnp.int32)
    fv = fv.at[0].set(0)
    pad = jnp.zeros((32 - NT,), jnp.int32)
    row6 = jnp.zeros((32,), jnp.int32).at[0].set(re[0]).at[1].set(
        re[1]).at[2].set(nr)
    P = jnp.stack([
        jnp.concatenate([gid, pad]),
        jnp.concatenate([actv, pad]),
        jnp.concatenate([rs, pad]),
        jnp.concatenate([par.astype(jnp.int32), pad]),
        jnp.concatenate([fe, pad]),
        jnp.concatenate([fv, pad]),
        row6,
        jnp.zeros((32,), jnp.int32),
    ])

    ys = pl.pallas_call(
        _ffn_body,
        grid_spec=pltpu.PrefetchScalarGridSpec(
            num_scalar_prefetch=1,
            grid=(NT,),
            in_specs=[
                pl.BlockSpec((BT, D), lambda r, g: (r, 0)),
                pl.BlockSpec(memory_space=pltpu.HBM),
                pl.BlockSpec((1, 1, FF), lambda r, g: (g[0, r], 0, 0)),
                pl.BlockSpec(memory_space=pltpu.HBM),
                pl.BlockSpec((1, 1, D), lambda r, g: (g[0, r], 0, 0)),
            ],
            out_specs=pl.BlockSpec((BT, D), lambda r, g: (r, 0)),
            scratch_shapes=[
                pltpu.VMEM((2, D, FF), jnp.float32),
                pltpu.VMEM((2, FF, D), jnp.float32),
                pltpu.SemaphoreType.DMA,
                pltpu.SemaphoreType.DMA,
                pltpu.SemaphoreType.DMA,
                pltpu.SemaphoreType.DMA,
            ],
        ),
        out_shape=jax.ShapeDtypeStruct((R, D), jnp.float32),
        compiler_params=pltpu.CompilerParams(
            dimension_semantics=("arbitrary",),
        ),
    )(P, xs, W1, b1r, W2, b2r)

    res = pl.kernel(
        _combine_body,
        out_type=jax.ShapeDtypeStruct((T, D), jnp.float32),
        mesh=mesh,
        scratch_types=[
            pltpu.VMEM((32,), jnp.int32),
            pltpu.VMEM((32,), jnp.int32),
            pltpu.VMEM((32, 16), jnp.float32),
            pltpu.VMEM((32, 16), jnp.float32),
            pltpu.VMEM((32, D), jnp.float32),
            pltpu.VMEM((32, D), jnp.float32),
            pltpu.VMEM((32, D), jnp.float32),
            pltpu.SemaphoreType.DMA,
        ],
    )(ys, pdst0, pdst1, s0m, s1m)

    return res.reshape(B, T, D)
